# baseline (speedup 1.0000x reference)
import functools

import jax
import jax.numpy as jnp
from jax import lax
from jax.experimental import pallas as pl
from jax.experimental.pallas import tpu as pltpu
from jax.experimental.pallas import tpu_sc as plsc

NC, NS, L = 2, 16, 16
NW = NC * NS
B, S, D = 1024, 200, 128
N = B * S
C = 64
NB = 4
EPS = 1e-5
BB = 16
NSPLIT = 4
HB = B // NSPLIT
N2 = N // NSPLIT

_mesh = plsc.VectorSubcoreMesh(core_axis_name="c", subcore_axis_name="s")


def _make_sc_body(n_tokens):
    tpw = n_tokens // NW
    nchunk = tpw // C
    loop_iters = nchunk // NB
    peel = nchunk - loop_iters * NB

    def _sc_body(x_ref, tok_ref, out_ref, idx_v, bufs0, bufs1, bufs2, bufs3,
                 pk0, pk1, pk2, pk3,
                 gsem0, gsem1, gsem2, gsem3, osem0, osem1, osem2, osem3):
        bufs = (bufs0, bufs1, bufs2, bufs3)
        pks = (pk0, pk1, pk2, pk3)
        gsems = (gsem0, gsem1, gsem2, gsem3)
        osems = (osem0, osem1, osem2, osem3)
        wid = lax.axis_index("s") * NC + lax.axis_index("c")
        base_tok = wid * tpw

        pltpu.sync_copy(x_ref.at[pl.ds(base_tok, tpw)], idx_v)

        def _gather(c, k):
            pltpu.async_copy(tok_ref.at[idx_v.at[pl.ds(c * C, C)]], bufs[k],
                             gsems[k])

        def _wait_out(k):
            pltpu.make_async_copy(
                pks[k], out_ref.at[pl.ds(base_tok, C)], osems[k]).wait()

        def _proc(c, k):
            buf, pk = bufs[k], pks[k]
            pltpu.make_async_copy(
                tok_ref.at[idx_v.at[pl.ds(0, C)]], bufs[k], gsems[k]).wait()

            def _packgrp(gi, carry):
                r0 = gi * 8
                for i in range(8):
                    r = r0 + i
                    for m in range(D // 32):
                        a = buf[r, pl.ds(m * L, L)]
                        b = buf[r, pl.ds(D // 2 + m * L, L)]
                        ba = lax.bitcast_convert_type(a, jnp.int32)
                        bb = lax.bitcast_convert_type(b, jnp.int32)
                        wa = lax.shift_right_logical(ba + 0x8000, 16)
                        wb = (bb + 0x8000) & jnp.int32(-65536)
                        pk[r, pl.ds(m * L, L)] = wa | wb
                return carry

            lax.fori_loop(0, C // 8, _packgrp, 0)
            pltpu.async_copy(pk, out_ref.at[pl.ds(base_tok + c * C, C)],
                             osems[k])

        _gather(0, 0)
        _gather(1, 1)

        def _step(c, u):
            ku2 = (u + 2) % NB

            @pl.when(c + 2 < nchunk)
            def _ga():
                @pl.when(c >= 2)
                def _wo():
                    _wait_out(ku2)
                _gather(c + 2, ku2)

            _proc(c, u)

        def _iterN(i, carry):
            for u in range(NB):
                _step(NB * i + u, u)
            return carry

        lax.fori_loop(0, loop_iters, _iterN, 0)
        for t in range(peel):
            c = loop_iters * NB + t
            _step(c, c % NB)
        for k in range(NB):
            _wait_out((nchunk - NB + k) % NB)

    return _sc_body


def _make_sc(n_tokens):
    return functools.partial(
        pl.kernel,
        out_type=jax.ShapeDtypeStruct((n_tokens, D // 2), jnp.int32),
        mesh=_mesh,
        scratch_types=(
            [pltpu.VMEM((n_tokens // NW,), jnp.int32)]
            + [pltpu.VMEM((C, D), jnp.float32)] * NB
            + [pltpu.VMEM((C, D // 2), jnp.int32)] * NB
            + [pltpu.SemaphoreType.DMA] * (2 * NB)
        ),
    )(_make_sc_body(n_tokens))


_sc_gather_half = _make_sc(N2)


def _ln_body(tok_ref, seg_ref, pos_ref, sege_ref, gam_ref, bet_ref, o_ref):
    H = D // 2
    t32 = tok_ref[...]
    ta = lax.bitcast_convert_type(t32 << 16, jnp.float32)
    tb = lax.bitcast_convert_type(t32 & jnp.int32(-65536), jnp.float32)
    g = seg_ref[...]
    pos = pos_ref[...]
    se = sege_ref[...]
    gb = g[:, :, None]

    def _half(t, sl):
        return (t + pos[None, :, sl] + se[0][None, None, sl]
                + gb * (se[1] - se[0])[None, None, sl])

    ha = _half(ta, slice(0, H))
    hb = _half(tb, slice(H, D))
    tot = (jnp.sum(ha, axis=-1, keepdims=True)
           + jnp.sum(hb, axis=-1, keepdims=True))
    mean = tot * (1.0 / D)
    ca = ha - mean
    cb = hb - mean
    var = (jnp.sum(ca * ca, axis=-1, keepdims=True)
           + jnp.sum(cb * cb, axis=-1, keepdims=True)) * (1.0 / D)
    rstd = lax.rsqrt(var + EPS)
    gam = gam_ref[...]
    bet = bet_ref[...]
    o_ref[:, :, 0:H] = ca * rstd * gam[None, None, 0:H] + bet[None, None, 0:H]
    o_ref[:, :, H:D] = cb * rstd * gam[None, None, H:D] + bet[None, None, H:D]


def _ln_body_alias(prev_ref, tok_ref, seg_ref, pos_ref, sege_ref, gam_ref,
                   bet_ref, o_ref):
    del prev_ref
    _ln_body(tok_ref, seg_ref, pos_ref, sege_ref, gam_ref, bet_ref, o_ref)


_ln_first = functools.partial(
    pl.pallas_call,
    out_shape=jax.ShapeDtypeStruct((B, S, D), jnp.float32),
    grid=(HB // BB,),
    in_specs=[
        pl.BlockSpec((BB, S, D // 2), lambda i: (i, 0, 0)),
        pl.BlockSpec((BB, S), lambda i: (i, 0)),
        pl.BlockSpec((S, D), lambda i: (0, 0)),
        pl.BlockSpec((2, D), lambda i: (0, 0)),
        pl.BlockSpec((D,), lambda i: (0,)),
        pl.BlockSpec((D,), lambda i: (0,)),
    ],
    out_specs=pl.BlockSpec((BB, S, D), lambda i: (i, 0, 0)),
)(_ln_body)

def _make_ln_next(part):
    off = part * (HB // BB)
    return functools.partial(
        pl.pallas_call,
        out_shape=jax.ShapeDtypeStruct((B, S, D), jnp.float32),
        grid=(HB // BB,),
        in_specs=[
            pl.BlockSpec((1, 8, D), lambda i: (0, 0, 0)),
            pl.BlockSpec((BB, S, D // 2), lambda i: (i, 0, 0)),
            pl.BlockSpec((BB, S), lambda i: (i, 0)),
            pl.BlockSpec((S, D), lambda i: (0, 0)),
            pl.BlockSpec((2, D), lambda i: (0, 0)),
            pl.BlockSpec((D,), lambda i: (0,)),
            pl.BlockSpec((D,), lambda i: (0,)),
        ],
        out_specs=pl.BlockSpec((BB, S, D),
                               lambda i, off=off: (i + off, 0, 0)),
        input_output_aliases={0: 0},
    )(_ln_body_alias)


_ln_next = [_make_ln_next(part) for part in range(1, NSPLIT)]


def kernel(x, seg, tok_embed, pos_embed, seg_embed, gamma, beta):
    x1 = x.reshape(N).astype(jnp.int32)
    segf = seg.astype(jnp.float32)
    pos = pos_embed[:S]
    rows = [_sc_gather_half(x1[k * N2:(k + 1) * N2], tok_embed)
            for k in range(NSPLIT)]
    out = _ln_first(rows[0].reshape(HB, S, D // 2), segf[:HB], pos, seg_embed,
                    gamma, beta)
    for k in range(1, NSPLIT):
        out = _ln_next[k - 1](out, rows[k].reshape(HB, S, D // 2),
                              segf[k * HB:(k + 1) * HB], pos, seg_embed,
                              gamma, beta)
    return out


# speedup vs baseline: 1.4160x; 1.4160x over previous
import functools

import jax
import jax.numpy as jnp
from jax import lax
from jax.experimental import pallas as pl
from jax.experimental.pallas import tpu as pltpu
from jax.experimental.pallas import tpu_sc as plsc

NC, NS, L = 2, 16, 16
NW = NC * NS
B, S, D = 1024, 200, 128
N = B * S
C = 64
NB = 4
EPS = 1e-5
BB = 16
NSPLIT = 4
HB = B // NSPLIT
N2 = N // NSPLIT

_mesh = plsc.VectorSubcoreMesh(core_axis_name="c", subcore_axis_name="s")


def _make_sc_body(n_tokens):
    tpw = n_tokens // NW
    nchunk = tpw // C
    loop_iters = nchunk // NB
    peel = nchunk - loop_iters * NB

    def _sc_body(x_ref, tok_ref, out_ref, idx_v, bufs0, bufs1, bufs2, bufs3,
                 gsem0, gsem1, gsem2, gsem3, osem0, osem1, osem2, osem3):
        bufs = (bufs0, bufs1, bufs2, bufs3)
        gsems = (gsem0, gsem1, gsem2, gsem3)
        osems = (osem0, osem1, osem2, osem3)
        wid = lax.axis_index("s") * NC + lax.axis_index("c")
        base_tok = wid * tpw

        pltpu.sync_copy(x_ref.at[pl.ds(base_tok, tpw)], idx_v)

        def _gather(c, k):
            pltpu.async_copy(tok_ref.at[idx_v.at[pl.ds(c * C, C)]], bufs[k],
                             gsems[k])

        def _wait_out(k):
            pltpu.make_async_copy(
                bufs[k], out_ref.at[pl.ds(base_tok, C)], osems[k]).wait()

        def _proc(c, k):
            pltpu.make_async_copy(
                tok_ref.at[idx_v.at[pl.ds(0, C)]], bufs[k], gsems[k]).wait()
            pltpu.async_copy(bufs[k], out_ref.at[pl.ds(base_tok + c * C, C)],
                             osems[k])

        _gather(0, 0)
        _gather(1, 1)

        def _step(c, u):
            ku2 = (u + 2) % NB

            @pl.when(c + 2 < nchunk)
            def _ga():
                @pl.when(c >= 2)
                def _wo():
                    _wait_out(ku2)
                _gather(c + 2, ku2)

            _proc(c, u)

        def _iterN(i, carry):
            for u in range(NB):
                _step(NB * i + u, u)
            return carry

        lax.fori_loop(0, loop_iters, _iterN, 0)
        for t in range(peel):
            c = loop_iters * NB + t
            _step(c, c % NB)
        for k in range(NB):
            _wait_out((nchunk - NB + k) % NB)

    return _sc_body


def _make_sc(n_tokens):
    return functools.partial(
        pl.kernel,
        out_type=jax.ShapeDtypeStruct((n_tokens, D), jnp.float32),
        mesh=_mesh,
        scratch_types=(
            [pltpu.VMEM((n_tokens // NW,), jnp.int32)]
            + [pltpu.VMEM((C, D), jnp.float32)] * NB
            + [pltpu.SemaphoreType.DMA] * (2 * NB)
        ),
    )(_make_sc_body(n_tokens))


_sc_gather_half = _make_sc(N2)


def _ln_body(tok_ref, seg_ref, pos_ref, sege_ref, gam_ref, bet_ref, o_ref):
    t = tok_ref[...]
    g = seg_ref[...]
    pos = pos_ref[...]
    se = sege_ref[...]
    h = (t + pos[None, :, :] + se[0][None, None, :]
         + g[:, :, None] * (se[1] - se[0])[None, None, :])
    mean = jnp.mean(h, axis=-1, keepdims=True)
    cen = h - mean
    var = jnp.mean(cen * cen, axis=-1, keepdims=True)
    o_ref[...] = (cen * lax.rsqrt(var + EPS) * gam_ref[...][None, None, :]
                  + bet_ref[...][None, None, :])


def _ln_body_alias(prev_ref, tok_ref, seg_ref, pos_ref, sege_ref, gam_ref,
                   bet_ref, o_ref):
    del prev_ref
    _ln_body(tok_ref, seg_ref, pos_ref, sege_ref, gam_ref, bet_ref, o_ref)


_ln_first = functools.partial(
    pl.pallas_call,
    out_shape=jax.ShapeDtypeStruct((B, S, D), jnp.float32),
    grid=(HB // BB,),
    in_specs=[
        pl.BlockSpec((BB, S, D), lambda i: (i, 0, 0)),
        pl.BlockSpec((BB, S), lambda i: (i, 0)),
        pl.BlockSpec((S, D), lambda i: (0, 0)),
        pl.BlockSpec((2, D), lambda i: (0, 0)),
        pl.BlockSpec((D,), lambda i: (0,)),
        pl.BlockSpec((D,), lambda i: (0,)),
    ],
    out_specs=pl.BlockSpec((BB, S, D), lambda i: (i, 0, 0)),
)(_ln_body)

def _make_ln_next(part):
    off = part * (HB // BB)
    return functools.partial(
        pl.pallas_call,
        out_shape=jax.ShapeDtypeStruct((B, S, D), jnp.float32),
        grid=(HB // BB,),
        in_specs=[
            pl.BlockSpec((1, 8, D), lambda i: (0, 0, 0)),
            pl.BlockSpec((BB, S, D), lambda i: (i, 0, 0)),
            pl.BlockSpec((BB, S), lambda i: (i, 0)),
            pl.BlockSpec((S, D), lambda i: (0, 0)),
            pl.BlockSpec((2, D), lambda i: (0, 0)),
            pl.BlockSpec((D,), lambda i: (0,)),
            pl.BlockSpec((D,), lambda i: (0,)),
        ],
        out_specs=pl.BlockSpec((BB, S, D),
                               lambda i, off=off: (i + off, 0, 0)),
        input_output_aliases={0: 0},
    )(_ln_body_alias)


_ln_next = [_make_ln_next(part) for part in range(1, NSPLIT)]


def kernel(x, seg, tok_embed, pos_embed, seg_embed, gamma, beta):
    x1 = x.reshape(N).astype(jnp.int32)
    segf = seg.astype(jnp.float32)
    pos = pos_embed[:S]
    rows = [_sc_gather_half(x1[k * N2:(k + 1) * N2], tok_embed)
            for k in range(NSPLIT)]
    out = _ln_first(rows[0].reshape(HB, S, D), segf[:HB], pos, seg_embed,
                    gamma, beta)
    for k in range(1, NSPLIT):
        out = _ln_next[k - 1](out, rows[k].reshape(HB, S, D),
                              segf[k * HB:(k + 1) * HB], pos, seg_embed,
                              gamma, beta)
    return out
